# baseline (speedup 1.0000x reference)
import jax
import jax.numpy as jnp
from jax.experimental import pallas as pl
from jax.experimental.pallas import tpu as pltpu

_SMOOTHING = 0.1
_IGNORE_INDEX = -100
_LANES = 128


def _loss_body(pred_ref, tgt_ref, tot_ref, gth_ref, *, bm, t_dim):
    j = pl.program_id(0)

    @pl.when(j == 0)
    def _init():
        tot_ref[0, 0] = 0.0
        gth_ref[0, 0] = 0.0

    data = pred_ref[...]
    tgt = tgt_ref[...]

    per_row = _LANES // t_dim
    m_iota = jax.lax.broadcasted_iota(jnp.int32, data.shape, 1)
    l_iota = jax.lax.broadcasted_iota(jnp.int32, data.shape, 2)
    cls = (j * bm + m_iota) * per_row + (l_iota // t_dim)

    masked = jnp.where(tgt != _IGNORE_INDEX, data, 0.0)
    tot_ref[0, 0] += jnp.sum(masked)
    gth_ref[0, 0] += jnp.sum(jnp.where(cls == tgt, data, 0.0))


def kernel(pred, target):
    B, C, T = pred.shape
    assert (C * T) % _LANES == 0 and _LANES % T == 0
    M = (C * T) // _LANES
    BM = 200
    assert M % BM == 0
    grid = M // BM

    pred_flat = pred.reshape(B, M, _LANES)
    tgt_tiled = jnp.tile(target, (1, _LANES // T)).reshape(B, 1, _LANES)

    import functools
    body = functools.partial(_loss_body, bm=BM, t_dim=T)
    tot, gth = pl.pallas_call(
        body,
        grid=(grid,),
        in_specs=[
            pl.BlockSpec((B, BM, _LANES), lambda j: (0, j, 0)),
            pl.BlockSpec((B, 1, _LANES), lambda j: (0, 0, 0)),
        ],
        out_specs=[
            pl.BlockSpec(memory_space=pltpu.SMEM),
            pl.BlockSpec(memory_space=pltpu.SMEM),
        ],
        out_shape=[
            jax.ShapeDtypeStruct((1, 1), jnp.float32),
            jax.ShapeDtypeStruct((1, 1), jnp.float32),
        ],
        compiler_params=pltpu.CompilerParams(
            dimension_semantics=("arbitrary",),
        ),
    )(pred_flat, tgt_tiled)

    fill = _SMOOTHING / (C - 1)
    conf = 1.0 - _SMOOTHING
    cnt = jnp.sum(target != _IGNORE_INDEX)
    denom = jnp.maximum(cnt, 1).astype(pred.dtype)
    loss = -(fill * tot[0, 0] + (conf - fill) * gth[0, 0]) / denom
    return jnp.where(cnt > 0, loss, jnp.zeros((), dtype=pred.dtype))


# speedup vs baseline: 1.0268x; 1.0268x over previous
import functools

import jax
import jax.numpy as jnp
from jax.experimental import pallas as pl
from jax.experimental.pallas import tpu as pltpu

_SMOOTHING = 0.1
_IGNORE_INDEX = -100
_LANES = 128


def _loss_body(pred_ref, qt_ref, maskf_ref, tot_ref, gth_ref,
               acc_tot, acc_gth, *, br, bpb, grid):
    j = pl.program_id(0)

    @pl.when(j == 0)
    def _init():
        acc_tot[...] = jnp.zeros_like(acc_tot)
        acc_gth[...] = jnp.zeros_like(acc_gth)

    data = pred_ref[...]
    qt = qt_ref[0]
    maskf = maskf_ref[0]

    qbase = (j % bpb) * br
    m_iota = qbase + jax.lax.broadcasted_iota(jnp.int32, (br, _LANES), 0)

    t_acc = acc_tot[...]
    g_acc = acc_gth[...]
    for i in range(br // 8):
        chunk = data[i * 8:(i + 1) * 8, :]
        t_acc = t_acc + chunk * maskf
        g_acc = g_acc + jnp.where(m_iota[i * 8:(i + 1) * 8, :] == qt, chunk, 0.0)
    acc_tot[...] = t_acc
    acc_gth[...] = g_acc

    @pl.when(j == grid - 1)
    def _fini():
        tot_ref[0, 0] = jnp.sum(acc_tot[...])
        gth_ref[0, 0] = jnp.sum(acc_gth[...])


def kernel(pred, target):
    B, C, T = pred.shape
    assert (C * T) % _LANES == 0 and _LANES % T == 0
    rows_per_b = (C * T) // _LANES
    R = B * rows_per_b
    BR = 5000
    assert rows_per_b % BR == 0 and BR % 8 == 0
    bpb = rows_per_b // BR
    grid = R // BR

    pred_flat = pred.reshape(R, _LANES)

    per_row = _LANES // T
    l = jnp.arange(_LANES, dtype=jnp.int32)
    t_l = l % T
    c_bl = jnp.take(target, t_l, axis=1).astype(jnp.int32)
    mask_bl = c_bl != _IGNORE_INDEX
    phase_ok = (c_bl % per_row) == (l // T)[None, :]
    q_bl = (c_bl * T + t_l[None, :]) // _LANES
    qt = jnp.where(mask_bl & phase_ok, q_bl, -1).astype(jnp.int32).reshape(B, 1, _LANES)
    maskf = mask_bl.astype(jnp.float32).reshape(B, 1, _LANES)

    body = functools.partial(_loss_body, br=BR, bpb=bpb, grid=grid)
    tot, gth = pl.pallas_call(
        body,
        grid=(grid,),
        in_specs=[
            pl.BlockSpec((BR, _LANES), lambda j: (j, 0)),
            pl.BlockSpec((1, 1, _LANES), lambda j, _bpb=bpb: (j // _bpb, 0, 0)),
            pl.BlockSpec((1, 1, _LANES), lambda j, _bpb=bpb: (j // _bpb, 0, 0)),
        ],
        out_specs=[
            pl.BlockSpec(memory_space=pltpu.SMEM),
            pl.BlockSpec(memory_space=pltpu.SMEM),
        ],
        out_shape=[
            jax.ShapeDtypeStruct((1, 1), jnp.float32),
            jax.ShapeDtypeStruct((1, 1), jnp.float32),
        ],
        scratch_shapes=[
            pltpu.VMEM((8, _LANES), jnp.float32),
            pltpu.VMEM((8, _LANES), jnp.float32),
        ],
        compiler_params=pltpu.CompilerParams(
            dimension_semantics=("arbitrary",),
        ),
    )(pred_flat, qt, maskf)

    fill = _SMOOTHING / (C - 1)
    conf = 1.0 - _SMOOTHING
    cnt = jnp.sum(target != _IGNORE_INDEX)
    denom = jnp.maximum(cnt, 1).astype(pred.dtype)
    loss = -(fill * tot[0, 0] + (conf - fill) * gth[0, 0]) / denom
    return jnp.where(cnt > 0, loss, jnp.zeros((), dtype=pred.dtype))


# speedup vs baseline: 1.0665x; 1.0387x over previous
import functools

import jax
import jax.numpy as jnp
from jax.experimental import pallas as pl
from jax.experimental.pallas import tpu as pltpu

_SMOOTHING = 0.1
_IGNORE_INDEX = -100


def _loss_body(pred_ref, qt_ref, maskf_ref, tot_ref, gth_ref,
               acc_tot, acc_gth, *, bc, t_dim, nb, nc):
    b = pl.program_id(0)
    j = pl.program_id(1)

    @pl.when((b == 0) & (j == 0))
    def _init():
        acc_tot[...] = jnp.zeros_like(acc_tot)
        acc_gth[...] = jnp.zeros_like(acc_gth)

    data = pred_ref[0]
    qt = qt_ref[0]
    maskf = maskf_ref[0]

    cls = j * bc + jax.lax.broadcasted_iota(jnp.int32, (bc, t_dim), 0)
    tot_part = jnp.sum(data, axis=0, keepdims=True) * maskf
    gth_part = jnp.sum(jnp.where(cls == qt, data, 0.0), axis=0, keepdims=True)
    acc_tot[0:1, :] += tot_part
    acc_gth[0:1, :] += gth_part

    @pl.when((b == nb - 1) & (j == nc - 1))
    def _fini():
        tot_ref[0, 0] = jnp.sum(acc_tot[0:1, :])
        gth_ref[0, 0] = jnp.sum(acc_gth[0:1, :])


def kernel(pred, target):
    B, C, T = pred.shape
    BC = 10000
    assert C % BC == 0 and BC % 8 == 0
    nc = C // BC

    tgt = target.astype(jnp.int32)
    mask = tgt != _IGNORE_INDEX
    qt = jnp.where(mask, tgt, -1).reshape(B, 1, T)
    maskf = mask.astype(jnp.float32).reshape(B, 1, T)

    body = functools.partial(_loss_body, bc=BC, t_dim=T, nb=B, nc=nc)
    tot, gth = pl.pallas_call(
        body,
        grid=(B, nc),
        in_specs=[
            pl.BlockSpec((1, BC, T), lambda b, j: (b, j, 0)),
            pl.BlockSpec((1, 1, T), lambda b, j: (b, 0, 0)),
            pl.BlockSpec((1, 1, T), lambda b, j: (b, 0, 0)),
        ],
        out_specs=[
            pl.BlockSpec(memory_space=pltpu.SMEM),
            pl.BlockSpec(memory_space=pltpu.SMEM),
        ],
        out_shape=[
            jax.ShapeDtypeStruct((1, 1), jnp.float32),
            jax.ShapeDtypeStruct((1, 1), jnp.float32),
        ],
        scratch_shapes=[
            pltpu.VMEM((8, T), jnp.float32),
            pltpu.VMEM((8, T), jnp.float32),
        ],
        compiler_params=pltpu.CompilerParams(
            dimension_semantics=("arbitrary", "arbitrary"),
        ),
    )(pred, qt, maskf)

    fill = _SMOOTHING / (C - 1)
    conf = 1.0 - _SMOOTHING
    cnt = jnp.sum(mask)
    denom = jnp.maximum(cnt, 1).astype(pred.dtype)
    loss = -(fill * tot[0, 0] + (conf - fill) * gth[0, 0]) / denom
    return jnp.where(cnt > 0, loss, jnp.zeros((), dtype=pred.dtype))
